# baseline (speedup 1.0000x reference)
import jax
import jax.numpy as jnp
from jax.experimental import pallas as pl

_SMOOTHING = 0.1
_NUM_CLASSES = 1000
_CONFIDENCE = 1.0 - _SMOOTHING
_LOW = _SMOOTHING / _NUM_CLASSES
_HIGH = _CONFIDENCE + _LOW

_BC = 512


def _smooth_kernel(lab_ref, out_ref):
    lab = lab_ref[0, 0, :]
    rows = jax.lax.broadcasted_iota(jnp.int32, (_NUM_CLASSES, _BC), 0)
    hit = rows == lab[None, :]
    out_ref[...] = jnp.where(hit, _HIGH, _LOW).astype(jnp.float32)


def kernel(labels):
    n = labels.shape[0]
    nb = n // _BC
    lab3 = labels.reshape(nb, 1, _BC)
    out_t = pl.pallas_call(
        _smooth_kernel,
        grid=(nb,),
        in_specs=[pl.BlockSpec((1, 1, _BC), lambda i: (i, 0, 0))],
        out_specs=pl.BlockSpec((_NUM_CLASSES, _BC), lambda i: (0, i)),
        out_shape=jax.ShapeDtypeStruct((_NUM_CLASSES, n), jnp.float32),
    )(lab3)
    return out_t.T


# speedup vs baseline: 1.3100x; 1.3100x over previous
import jax
import jax.numpy as jnp
from jax.experimental import pallas as pl

_SMOOTHING = 0.1
_NUM_CLASSES = 1000
_CONFIDENCE = 1.0 - _SMOOTHING
_LOW = _SMOOTHING / _NUM_CLASSES
_HIGH = _CONFIDENCE + _LOW

_BC = 1024


def _smooth_kernel(lab_ref, out_ref):
    lab = lab_ref[0, 0, :]
    rows = jax.lax.broadcasted_iota(jnp.int32, (_NUM_CLASSES, _BC), 0)
    hit = rows == lab[None, :]
    out_ref[...] = jnp.where(hit, _HIGH, _LOW).astype(jnp.float32)


def kernel(labels):
    n = labels.shape[0]
    nb = n // _BC
    lab3 = labels.reshape(nb, 1, _BC)
    out_t = pl.pallas_call(
        _smooth_kernel,
        grid=(nb,),
        in_specs=[pl.BlockSpec((1, 1, _BC), lambda i: (i, 0, 0))],
        out_specs=pl.BlockSpec((_NUM_CLASSES, _BC), lambda i: (0, i)),
        out_shape=jax.ShapeDtypeStruct((_NUM_CLASSES, n), jnp.float32),
    )(lab3)
    return out_t.T
